# Initial kernel scaffold; baseline (speedup 1.0000x reference)
import jax
import jax.numpy as jnp
from jax.experimental import pallas as pl


def kernel(x, edge_index, W1, b1, W2, b2, W3, b3, Wc, bc):
    raise NotImplementedError("write your pallas kernel here")


# speedup vs baseline: 18.6777x; 18.6777x over previous
import functools

import jax
import jax.numpy as jnp
from jax import lax
from jax.experimental import pallas as pl
from jax.experimental.pallas import tpu as pltpu
from jax.experimental.pallas import tpu_sc as plsc

N = 10000
D = 128
E = 320000
NC, NS = 2, 16
NW = NC * NS
NP = 10240
RPW = NP // NS
H = 8
CHUNK = 128
CPW = 80
EP = NW * CPW * CHUNK
PAD_NODE = N + 16

BLK = 1024
GRID = NP // BLK


def _sc_mesh():
    return plsc.VectorSubcoreMesh(
        core_axis_name="c", subcore_axis_name="s",
        num_cores=NC, num_subcores=NS)


_SC_PARAMS = pltpu.CompilerParams(use_tc_tiling_on_sc=False)


@functools.lru_cache(maxsize=None)
def _sc_agg():
    @functools.partial(
        pl.kernel,
        out_type=jax.ShapeDtypeStruct((NC, NP, H), jnp.float32),
        mesh=_sc_mesh(),
        scratch_types=[
            pltpu.VMEM((CHUNK,), jnp.int32),
            pltpu.VMEM((CHUNK,), jnp.int32),
            pltpu.VMEM((CHUNK, H), jnp.float32),
            pltpu.VMEM_SHARED((NP, H), jnp.float32),
        ],
        compiler_params=_SC_PARAMS,
    )
    def agg(src_hbm, dst_hbm, g_hbm, z_hbm, out_hbm, srcv, dstv, rows, acc):
        c = lax.axis_index("c")
        s = lax.axis_index("s")
        w = c * NS + s
        pltpu.sync_copy(z_hbm.at[pl.ds(s * RPW, RPW)],
                        acc.at[pl.ds(s * RPW, RPW)])
        plsc.subcore_barrier()

        @pl.loop(0, CPW)
        def body(j):
            off = pl.multiple_of((w * CPW + j) * CHUNK, CHUNK)
            pltpu.sync_copy(src_hbm.at[pl.ds(off, CHUNK)], srcv)
            pltpu.sync_copy(dst_hbm.at[pl.ds(off, CHUNK)], dstv)
            pltpu.sync_copy(g_hbm.at[srcv], rows)
            pltpu.sync_copy(rows, acc.at[dstv], add=True)

        plsc.subcore_barrier()
        pltpu.sync_copy(acc.at[pl.ds(s * RPW, RPW)],
                        out_hbm.at[c, pl.ds(s * RPW, RPW)])

    return agg


@functools.lru_cache(maxsize=None)
def _sc_deg():
    @functools.partial(
        pl.kernel,
        out_type=jax.ShapeDtypeStruct((NC, NP, H), jnp.float32),
        mesh=_sc_mesh(),
        scratch_types=[
            pltpu.VMEM((CHUNK,), jnp.int32),
            pltpu.VMEM((CHUNK, H), jnp.float32),
            pltpu.VMEM_SHARED((NP, H), jnp.float32),
        ],
        compiler_params=_SC_PARAMS,
    )
    def deg(dst_hbm, ones_hbm, z_hbm, out_hbm, dstv, ones_v, acc):
        c = lax.axis_index("c")
        s = lax.axis_index("s")
        w = c * NS + s
        pltpu.sync_copy(ones_hbm, ones_v)
        pltpu.sync_copy(z_hbm.at[pl.ds(s * RPW, RPW)],
                        acc.at[pl.ds(s * RPW, RPW)])
        plsc.subcore_barrier()

        @pl.loop(0, CPW)
        def body(j):
            off = pl.multiple_of((w * CPW + j) * CHUNK, CHUNK)
            pltpu.sync_copy(dst_hbm.at[pl.ds(off, CHUNK)], dstv)
            pltpu.sync_copy(ones_v, acc.at[dstv], add=True)

        plsc.subcore_barrier()
        pltpu.sync_copy(acc.at[pl.ds(s * RPW, RPW)],
                        out_hbm.at[c, pl.ds(s * RPW, RPW)])

    return deg


def _tc1(xp, W1, degs):
    def body(x_ref, w_ref, deg_ref, g_ref, dinv_ref):
        deg = deg_ref[0, :, 0:1] + deg_ref[1, :, 0:1] + 1.0
        dinv = lax.rsqrt(deg)
        h = jnp.dot(x_ref[...], w_ref[...],
                    preferred_element_type=jnp.float32)
        g_ref[...] = jnp.concatenate(
            [h * dinv, jnp.zeros((BLK, H - 4), jnp.float32)], axis=1)
        dinv_ref[...] = dinv

    return pl.pallas_call(
        body,
        grid=(GRID,),
        in_specs=[
            pl.BlockSpec((BLK, D), lambda i: (i, 0)),
            pl.BlockSpec((D, 4), lambda i: (0, 0)),
            pl.BlockSpec((NC, BLK, H), lambda i: (0, i, 0)),
        ],
        out_specs=[
            pl.BlockSpec((BLK, H), lambda i: (i, 0)),
            pl.BlockSpec((BLK, 1), lambda i: (i, 0)),
        ],
        out_shape=[
            jax.ShapeDtypeStruct((NP, H), jnp.float32),
            jax.ShapeDtypeStruct((NP, 1), jnp.float32),
        ],
    )(xp, W1, degs)


def _tc_mid(acc, g, dinv, b, W):
    hin = W.shape[0]
    hout = W.shape[1]

    def body(acc_ref, g_ref, dinv_ref, b_ref, w_ref, out_ref):
        dv = dinv_ref[...]
        t = ((acc_ref[0, :, :hin] + acc_ref[1, :, :hin] + g_ref[:, :hin]) * dv
             + b_ref[...])
        xx = jnp.tanh(t)
        h = jnp.dot(xx, w_ref[...], preferred_element_type=jnp.float32)
        out_ref[...] = jnp.concatenate(
            [h * dv, jnp.zeros((BLK, H - hout), jnp.float32)], axis=1)

    return pl.pallas_call(
        body,
        grid=(GRID,),
        in_specs=[
            pl.BlockSpec((NC, BLK, H), lambda i: (0, i, 0)),
            pl.BlockSpec((BLK, H), lambda i: (i, 0)),
            pl.BlockSpec((BLK, 1), lambda i: (i, 0)),
            pl.BlockSpec((1, hin), lambda i: (0, 0)),
            pl.BlockSpec((hin, hout), lambda i: (0, 0)),
        ],
        out_specs=pl.BlockSpec((BLK, H), lambda i: (i, 0)),
        out_shape=jax.ShapeDtypeStruct((NP, H), jnp.float32),
    )(acc, g, dinv, b, W)


def _tc_final(acc, g, dinv, b, Wc, bc):
    hin = Wc.shape[0]
    c = Wc.shape[1]

    def body(acc_ref, g_ref, dinv_ref, b_ref, wc_ref, bc_ref,
             out_ref, h_ref):
        dv = dinv_ref[...]
        t = ((acc_ref[0, :, :hin] + acc_ref[1, :, :hin] + g_ref[:, :hin]) * dv
             + b_ref[...])
        x3 = jnp.tanh(t)
        out_ref[...] = jnp.dot(x3, wc_ref[...],
                               preferred_element_type=jnp.float32) + bc_ref[...]
        h_ref[...] = x3

    return pl.pallas_call(
        body,
        grid=(GRID,),
        in_specs=[
            pl.BlockSpec((NC, BLK, H), lambda i: (0, i, 0)),
            pl.BlockSpec((BLK, H), lambda i: (i, 0)),
            pl.BlockSpec((BLK, 1), lambda i: (i, 0)),
            pl.BlockSpec((1, hin), lambda i: (0, 0)),
            pl.BlockSpec((hin, c), lambda i: (0, 0)),
            pl.BlockSpec((1, c), lambda i: (0, 0)),
        ],
        out_specs=[
            pl.BlockSpec((BLK, c), lambda i: (i, 0)),
            pl.BlockSpec((BLK, hin), lambda i: (i, 0)),
        ],
        out_shape=[
            jax.ShapeDtypeStruct((NP, c), jnp.float32),
            jax.ShapeDtypeStruct((NP, hin), jnp.float32),
        ],
    )(acc, g, dinv, b, Wc, bc)


def kernel(x, edge_index, W1, b1, W2, b2, W3, b3, Wc, bc):
    pad = jnp.full((EP - E,), PAD_NODE, jnp.int32)
    srcp = jnp.concatenate([edge_index[0], pad])
    dstp = jnp.concatenate([edge_index[1], pad])
    xp = jnp.pad(x, ((0, NP - N), (0, 0)))
    z8 = jnp.zeros((NP, H), jnp.float32)
    ones = jnp.ones((CHUNK, H), jnp.float32)

    degs = _sc_deg()(dstp, ones, z8)
    g1, dinv = _tc1(xp, W1, degs)
    acc1 = _sc_agg()(srcp, dstp, g1, z8)
    g2 = _tc_mid(acc1, g1, dinv, b1.reshape(1, -1), W2)
    acc2 = _sc_agg()(srcp, dstp, g2, z8)
    g3 = _tc_mid(acc2, g2, dinv, b2.reshape(1, -1), W3)
    acc3 = _sc_agg()(srcp, dstp, g3, z8)
    out, h = _tc_final(acc3, g3, dinv, b3.reshape(1, -1),
                       Wc, bc.reshape(1, -1))
    return out[:N], h[:N]


# speedup vs baseline: 37.1053x; 1.9866x over previous
import functools

import jax
import jax.numpy as jnp
from jax import lax
from jax.experimental import pallas as pl
from jax.experimental.pallas import tpu as pltpu
from jax.experimental.pallas import tpu_sc as plsc

N = 10000
D = 128
E = 320000
NC, NS = 2, 16
NW = NC * NS
NP = 10240
RPW = NP // NS
H = 8
CHUNK = 128
CPW = 80
M = NW * CPW
EP = M * CHUNK
PAD_NODE = N + 16
SETS = 8

BLK = 1024
GRID = NP // BLK


def _sc_mesh():
    return plsc.VectorSubcoreMesh(
        core_axis_name="c", subcore_axis_name="s",
        num_cores=NC, num_subcores=NS)


_SC_PARAMS = pltpu.CompilerParams(use_tc_tiling_on_sc=False)


@functools.lru_cache(maxsize=None)
def _sc_agg():
    @functools.partial(
        pl.kernel,
        out_type=jax.ShapeDtypeStruct((NC, NP, H), jnp.float32),
        mesh=_sc_mesh(),
        scratch_types=(
            [pltpu.VMEM((2, CHUNK), jnp.int32) for _ in range(SETS)]
            + [pltpu.VMEM((CHUNK, H), jnp.float32) for _ in range(SETS)]
            + [pltpu.SemaphoreType.DMA] * (2 * SETS)
            + [pltpu.VMEM_SHARED((NP, H), jnp.float32)]
        ),
        compiler_params=_SC_PARAMS,
    )
    def agg(sd_hbm, g_hbm, z_hbm, out_hbm, *scr):
        sdb = scr[:SETS]
        rows = scr[SETS:2 * SETS]
        gsem = scr[2 * SETS:3 * SETS]
        ssem = scr[3 * SETS:4 * SETS]
        acc = scr[4 * SETS]
        c = lax.axis_index("c")
        s = lax.axis_index("s")
        w = c * NS + s
        pltpu.sync_copy(z_hbm.at[pl.ds(s * RPW, RPW)],
                        acc.at[pl.ds(s * RPW, RPW)])
        plsc.subcore_barrier()

        @pl.loop(0, CPW, step=SETS)
        def body(j):
            descs = []
            for k in range(SETS):
                @pl.when(j > 0)
                def _(k=k):
                    pltpu.make_async_copy(
                        rows[k], acc.at[sdb[k].at[1]], ssem[k]).wait()
                descs.append(pltpu.async_copy(
                    sd_hbm.at[w * CPW + j + k], sdb[k], gsem[k]))
            for k in range(SETS):
                descs[k].wait()
                pltpu.async_copy(g_hbm.at[sdb[k].at[0]], rows[k], gsem[k])
            for k in range(SETS):
                pltpu.make_async_copy(
                    g_hbm.at[sdb[k].at[0]], rows[k], gsem[k]).wait()
                pltpu.async_copy(
                    rows[k], acc.at[sdb[k].at[1]], ssem[k], add=True)

        for k in range(SETS):
            pltpu.make_async_copy(
                rows[k], acc.at[sdb[k].at[1]], ssem[k]).wait()
        plsc.subcore_barrier()
        pltpu.sync_copy(acc.at[pl.ds(s * RPW, RPW)],
                        out_hbm.at[c, pl.ds(s * RPW, RPW)])

    return agg


@functools.lru_cache(maxsize=None)
def _sc_deg():
    @functools.partial(
        pl.kernel,
        out_type=jax.ShapeDtypeStruct((NC, NP, H), jnp.float32),
        mesh=_sc_mesh(),
        scratch_types=(
            [pltpu.VMEM((2, CHUNK), jnp.int32) for _ in range(SETS)]
            + [pltpu.SemaphoreType.DMA] * SETS
            + [pltpu.VMEM((CHUNK, H), jnp.float32),
               pltpu.VMEM_SHARED((NP, H), jnp.float32)]
        ),
        compiler_params=_SC_PARAMS,
    )
    def deg(sd_hbm, ones_hbm, z_hbm, out_hbm, *scr):
        sdb = scr[:SETS]
        ssem = scr[SETS:2 * SETS]
        ones_v = scr[2 * SETS]
        acc = scr[2 * SETS + 1]
        c = lax.axis_index("c")
        s = lax.axis_index("s")
        w = c * NS + s
        pltpu.sync_copy(ones_hbm, ones_v)
        pltpu.sync_copy(z_hbm.at[pl.ds(s * RPW, RPW)],
                        acc.at[pl.ds(s * RPW, RPW)])
        plsc.subcore_barrier()

        @pl.loop(0, CPW, step=SETS)
        def body(j):
            descs = []
            for k in range(SETS):
                @pl.when(j > 0)
                def _(k=k):
                    pltpu.make_async_copy(
                        ones_v, acc.at[sdb[k].at[1]], ssem[k]).wait()
                descs.append(pltpu.async_copy(
                    sd_hbm.at[w * CPW + j + k], sdb[k], ssem[k]))
            for k in range(SETS):
                descs[k].wait()
                pltpu.async_copy(
                    ones_v, acc.at[sdb[k].at[1]], ssem[k], add=True)

        for k in range(SETS):
            pltpu.make_async_copy(
                ones_v, acc.at[sdb[k].at[1]], ssem[k]).wait()
        plsc.subcore_barrier()
        pltpu.sync_copy(acc.at[pl.ds(s * RPW, RPW)],
                        out_hbm.at[c, pl.ds(s * RPW, RPW)])

    return deg


def _tc1(xp, W1, degs):
    def body(x_ref, w_ref, deg_ref, g_ref, dinv_ref):
        deg = deg_ref[0, :, 0:1] + deg_ref[1, :, 0:1] + 1.0
        dinv = lax.rsqrt(deg)
        h = jnp.dot(x_ref[...], w_ref[...],
                    preferred_element_type=jnp.float32)
        g_ref[...] = jnp.concatenate(
            [h * dinv, jnp.zeros((BLK, H - 4), jnp.float32)], axis=1)
        dinv_ref[...] = dinv

    return pl.pallas_call(
        body,
        grid=(GRID,),
        in_specs=[
            pl.BlockSpec((BLK, D), lambda i: (i, 0)),
            pl.BlockSpec((D, 4), lambda i: (0, 0)),
            pl.BlockSpec((NC, BLK, H), lambda i: (0, i, 0)),
        ],
        out_specs=[
            pl.BlockSpec((BLK, H), lambda i: (i, 0)),
            pl.BlockSpec((BLK, 1), lambda i: (i, 0)),
        ],
        out_shape=[
            jax.ShapeDtypeStruct((NP, H), jnp.float32),
            jax.ShapeDtypeStruct((NP, 1), jnp.float32),
        ],
    )(xp, W1, degs)


def _tc_mid(acc, g, dinv, b, W):
    hin = W.shape[0]
    hout = W.shape[1]

    def body(acc_ref, g_ref, dinv_ref, b_ref, w_ref, out_ref):
        dv = dinv_ref[...]
        t = ((acc_ref[0, :, :hin] + acc_ref[1, :, :hin] + g_ref[:, :hin]) * dv
             + b_ref[...])
        xx = jnp.tanh(t)
        h = jnp.dot(xx, w_ref[...], preferred_element_type=jnp.float32)
        out_ref[...] = jnp.concatenate(
            [h * dv, jnp.zeros((BLK, H - hout), jnp.float32)], axis=1)

    return pl.pallas_call(
        body,
        grid=(GRID,),
        in_specs=[
            pl.BlockSpec((NC, BLK, H), lambda i: (0, i, 0)),
            pl.BlockSpec((BLK, H), lambda i: (i, 0)),
            pl.BlockSpec((BLK, 1), lambda i: (i, 0)),
            pl.BlockSpec((1, hin), lambda i: (0, 0)),
            pl.BlockSpec((hin, hout), lambda i: (0, 0)),
        ],
        out_specs=pl.BlockSpec((BLK, H), lambda i: (i, 0)),
        out_shape=jax.ShapeDtypeStruct((NP, H), jnp.float32),
    )(acc, g, dinv, b, W)


def _tc_final(acc, g, dinv, b, Wc, bc):
    hin = Wc.shape[0]
    c = Wc.shape[1]

    def body(acc_ref, g_ref, dinv_ref, b_ref, wc_ref, bc_ref,
             out_ref, h_ref):
        dv = dinv_ref[...]
        t = ((acc_ref[0, :, :hin] + acc_ref[1, :, :hin] + g_ref[:, :hin]) * dv
             + b_ref[...])
        x3 = jnp.tanh(t)
        out_ref[...] = jnp.dot(x3, wc_ref[...],
                               preferred_element_type=jnp.float32) + bc_ref[...]
        h_ref[...] = x3

    return pl.pallas_call(
        body,
        grid=(GRID,),
        in_specs=[
            pl.BlockSpec((NC, BLK, H), lambda i: (0, i, 0)),
            pl.BlockSpec((BLK, H), lambda i: (i, 0)),
            pl.BlockSpec((BLK, 1), lambda i: (i, 0)),
            pl.BlockSpec((1, hin), lambda i: (0, 0)),
            pl.BlockSpec((hin, c), lambda i: (0, 0)),
            pl.BlockSpec((1, c), lambda i: (0, 0)),
        ],
        out_specs=[
            pl.BlockSpec((BLK, c), lambda i: (i, 0)),
            pl.BlockSpec((BLK, hin), lambda i: (i, 0)),
        ],
        out_shape=[
            jax.ShapeDtypeStruct((NP, c), jnp.float32),
            jax.ShapeDtypeStruct((NP, hin), jnp.float32),
        ],
    )(acc, g, dinv, b, Wc, bc)


def kernel(x, edge_index, W1, b1, W2, b2, W3, b3, Wc, bc):
    pad = jnp.full((EP - E,), PAD_NODE, jnp.int32)
    srcp = jnp.concatenate([edge_index[0], pad]).reshape(M, CHUNK)
    dstp = jnp.concatenate([edge_index[1], pad]).reshape(M, CHUNK)
    sd = jnp.stack([srcp, dstp], axis=1)
    xp = jnp.pad(x, ((0, NP - N), (0, 0)))
    z8 = jnp.zeros((NP, H), jnp.float32)
    ones = jnp.ones((CHUNK, H), jnp.float32)

    degs = _sc_deg()(sd, ones, z8)
    g1, dinv = _tc1(xp, W1, degs)
    acc1 = _sc_agg()(sd, g1, z8)
    g2 = _tc_mid(acc1, g1, dinv, b1.reshape(1, -1), W2)
    acc2 = _sc_agg()(sd, g2, z8)
    g3 = _tc_mid(acc2, g2, dinv, b2.reshape(1, -1), W3)
    acc3 = _sc_agg()(sd, g3, z8)
    out, h = _tc_final(acc3, g3, dinv, b3.reshape(1, -1),
                       Wc, bc.reshape(1, -1))
    return out[:N], h[:N]


# speedup vs baseline: 48.2771x; 1.3011x over previous
import functools

import jax
import jax.numpy as jnp
from jax import lax
from jax.experimental import pallas as pl
from jax.experimental.pallas import tpu as pltpu
from jax.experimental.pallas import tpu_sc as plsc

N = 10000
D = 128
E = 320000
NC, NS = 2, 16
NW = NC * NS
NP = 10240
RPW = NP // NS
H = 8
CHUNK = 128
CPW = 80
M = NW * CPW
EP = M * CHUNK
SETS = 8

BLK = 1024
GRID = NP // BLK


def _sc_mesh():
    return plsc.VectorSubcoreMesh(
        core_axis_name="c", subcore_axis_name="s",
        num_cores=NC, num_subcores=NS)


_SC_PARAMS = pltpu.CompilerParams(use_tc_tiling_on_sc=False)


@functools.lru_cache(maxsize=None)
def _sc_agg():
    @functools.partial(
        pl.kernel,
        out_type=jax.ShapeDtypeStruct((NC, NP, H), jnp.float32),
        mesh=_sc_mesh(),
        scratch_types=(
            [pltpu.VMEM((2, CHUNK), jnp.int32) for _ in range(SETS)]
            + [pltpu.VMEM((CHUNK, H), jnp.float32) for _ in range(SETS)]
            + [pltpu.SemaphoreType.DMA] * (2 * SETS)
            + [pltpu.VMEM_SHARED((NP, H), jnp.float32)]
        ),
        compiler_params=_SC_PARAMS,
    )
    def agg(sd_hbm, g_hbm, z_hbm, out_hbm, *scr):
        sdb = scr[:SETS]
        rows = scr[SETS:2 * SETS]
        gsem = scr[2 * SETS:3 * SETS]
        ssem = scr[3 * SETS:4 * SETS]
        acc = scr[4 * SETS]
        c = lax.axis_index("c")
        s = lax.axis_index("s")
        w = c * NS + s
        pltpu.sync_copy(z_hbm.at[pl.ds(s * RPW, RPW)],
                        acc.at[pl.ds(s * RPW, RPW)])
        plsc.subcore_barrier()

        @pl.loop(0, CPW, step=SETS)
        def body(j):
            descs = []
            for k in range(SETS):
                @pl.when(j > 0)
                def _(k=k):
                    pltpu.make_async_copy(
                        rows[k], acc.at[sdb[k].at[1]], ssem[k]).wait()
                descs.append(pltpu.async_copy(
                    sd_hbm.at[w * CPW + j + k], sdb[k], gsem[k]))
            for k in range(SETS):
                descs[k].wait()
                pltpu.async_copy(g_hbm.at[sdb[k].at[0]], rows[k], gsem[k])
            for k in range(SETS):
                pltpu.make_async_copy(
                    g_hbm.at[sdb[k].at[0]], rows[k], gsem[k]).wait()
                pltpu.async_copy(
                    rows[k], acc.at[sdb[k].at[1]], ssem[k], add=True)

        for k in range(SETS):
            pltpu.make_async_copy(
                rows[k], acc.at[sdb[k].at[1]], ssem[k]).wait()
        plsc.subcore_barrier()
        pltpu.sync_copy(acc.at[pl.ds(s * RPW, RPW)],
                        out_hbm.at[c, pl.ds(s * RPW, RPW)])

    return agg


@functools.lru_cache(maxsize=None)
def _sc_deg():
    @functools.partial(
        pl.kernel,
        out_type=jax.ShapeDtypeStruct((NC, NP, H), jnp.float32),
        mesh=_sc_mesh(),
        scratch_types=(
            [pltpu.VMEM((2, CHUNK), jnp.int32) for _ in range(SETS)]
            + [pltpu.SemaphoreType.DMA] * SETS
            + [pltpu.VMEM((CHUNK, H), jnp.float32),
               pltpu.VMEM_SHARED((NP, H), jnp.float32)]
        ),
        compiler_params=_SC_PARAMS,
    )
    def deg(sd_hbm, ones_hbm, z_hbm, out_hbm, *scr):
        sdb = scr[:SETS]
        ssem = scr[SETS:2 * SETS]
        ones_v = scr[2 * SETS]
        acc = scr[2 * SETS + 1]
        c = lax.axis_index("c")
        s = lax.axis_index("s")
        w = c * NS + s
        pltpu.sync_copy(ones_hbm, ones_v)
        pltpu.sync_copy(z_hbm.at[pl.ds(s * RPW, RPW)],
                        acc.at[pl.ds(s * RPW, RPW)])
        plsc.subcore_barrier()

        @pl.loop(0, CPW, step=SETS)
        def body(j):
            descs = []
            for k in range(SETS):
                @pl.when(j > 0)
                def _(k=k):
                    pltpu.make_async_copy(
                        ones_v, acc.at[sdb[k].at[1]], ssem[k]).wait()
                descs.append(pltpu.async_copy(
                    sd_hbm.at[w * CPW + j + k], sdb[k], ssem[k]))
            for k in range(SETS):
                descs[k].wait()
                pltpu.async_copy(
                    ones_v, acc.at[sdb[k].at[1]], ssem[k], add=True)

        for k in range(SETS):
            pltpu.make_async_copy(
                ones_v, acc.at[sdb[k].at[1]], ssem[k]).wait()
        plsc.subcore_barrier()
        pltpu.sync_copy(acc.at[pl.ds(s * RPW, RPW)],
                        out_hbm.at[c, pl.ds(s * RPW, RPW)])

    return deg


def _tc1(xp, W1, degs):
    def body(x_ref, w_ref, deg_ref, g_ref, dinv_ref):
        deg = deg_ref[0, :, 0:1] + deg_ref[1, :, 0:1] + 1.0
        dinv = lax.rsqrt(deg)
        h = jnp.dot(x_ref[...], w_ref[...],
                    preferred_element_type=jnp.float32)
        g_ref[...] = jnp.concatenate(
            [h * dinv, jnp.zeros((BLK, H - 4), jnp.float32)], axis=1)
        dinv_ref[...] = dinv

    return pl.pallas_call(
        body,
        grid=(GRID,),
        in_specs=[
            pl.BlockSpec((BLK, D), lambda i: (i, 0)),
            pl.BlockSpec((D, 4), lambda i: (0, 0)),
            pl.BlockSpec((NC, BLK, H), lambda i: (0, i, 0)),
        ],
        out_specs=[
            pl.BlockSpec((BLK, H), lambda i: (i, 0)),
            pl.BlockSpec((BLK, 1), lambda i: (i, 0)),
        ],
        out_shape=[
            jax.ShapeDtypeStruct((NP, H), jnp.float32),
            jax.ShapeDtypeStruct((NP, 1), jnp.float32),
        ],
    )(xp, W1, degs)


def _tc_mid(acc, g, dinv, b, W):
    hin = W.shape[0]
    hout = W.shape[1]

    def body(acc_ref, g_ref, dinv_ref, b_ref, w_ref, out_ref):
        dv = dinv_ref[...]
        t = ((acc_ref[0, :, :hin] + acc_ref[1, :, :hin] + g_ref[:, :hin]) * dv
             + b_ref[...])
        xx = jnp.tanh(t)
        h = jnp.dot(xx, w_ref[...], preferred_element_type=jnp.float32)
        out_ref[...] = jnp.concatenate(
            [h * dv, jnp.zeros((BLK, H - hout), jnp.float32)], axis=1)

    return pl.pallas_call(
        body,
        grid=(GRID,),
        in_specs=[
            pl.BlockSpec((NC, BLK, H), lambda i: (0, i, 0)),
            pl.BlockSpec((BLK, H), lambda i: (i, 0)),
            pl.BlockSpec((BLK, 1), lambda i: (i, 0)),
            pl.BlockSpec((1, hin), lambda i: (0, 0)),
            pl.BlockSpec((hin, hout), lambda i: (0, 0)),
        ],
        out_specs=pl.BlockSpec((BLK, H), lambda i: (i, 0)),
        out_shape=jax.ShapeDtypeStruct((NP, H), jnp.float32),
    )(acc, g, dinv, b, W)


def _tc_final(acc, g, dinv, b, Wc, bc):
    hin = Wc.shape[0]
    c = Wc.shape[1]

    def body(acc_ref, g_ref, dinv_ref, b_ref, wc_ref, bc_ref,
             out_ref, h_ref):
        dv = dinv_ref[...]
        t = ((acc_ref[0, :, :hin] + acc_ref[1, :, :hin] + g_ref[:, :hin]) * dv
             + b_ref[...])
        x3 = jnp.tanh(t)
        out_ref[...] = jnp.dot(x3, wc_ref[...],
                               preferred_element_type=jnp.float32) + bc_ref[...]
        h_ref[...] = x3

    return pl.pallas_call(
        body,
        grid=(GRID,),
        in_specs=[
            pl.BlockSpec((NC, BLK, H), lambda i: (0, i, 0)),
            pl.BlockSpec((BLK, H), lambda i: (i, 0)),
            pl.BlockSpec((BLK, 1), lambda i: (i, 0)),
            pl.BlockSpec((1, hin), lambda i: (0, 0)),
            pl.BlockSpec((hin, c), lambda i: (0, 0)),
            pl.BlockSpec((1, c), lambda i: (0, 0)),
        ],
        out_specs=[
            pl.BlockSpec((BLK, c), lambda i: (i, 0)),
            pl.BlockSpec((BLK, hin), lambda i: (i, 0)),
        ],
        out_shape=[
            jax.ShapeDtypeStruct((NP, c), jnp.float32),
            jax.ShapeDtypeStruct((NP, hin), jnp.float32),
        ],
    )(acc, g, dinv, b, Wc, bc)


def kernel(x, edge_index, W1, b1, W2, b2, W3, b3, Wc, bc):
    epw = E // NW
    ppw = CPW * CHUNK - epw
    pad_src = jnp.broadcast_to(
        N + 112 + (jnp.arange(ppw, dtype=jnp.int32) % 128), (NW, ppw))
    pad_dst = jnp.broadcast_to(
        N + 112 + ((jnp.arange(ppw, dtype=jnp.int32) + 64) % 128), (NW, ppw))
    srcp = jnp.concatenate(
        [edge_index[0].reshape(NW, epw), pad_src], axis=1).reshape(M, CHUNK)
    dstp = jnp.concatenate(
        [edge_index[1].reshape(NW, epw), pad_dst], axis=1).reshape(M, CHUNK)
    sd = jnp.stack([srcp, dstp], axis=1)
    xp = jnp.pad(x, ((0, NP - N), (0, 0)))
    z8 = jnp.zeros((NP, H), jnp.float32)
    ones = jnp.ones((CHUNK, H), jnp.float32)

    degs = _sc_deg()(sd, ones, z8)
    g1, dinv = _tc1(xp, W1, degs)
    acc1 = _sc_agg()(sd, g1, z8)
    g2 = _tc_mid(acc1, g1, dinv, b1.reshape(1, -1), W2)
    acc2 = _sc_agg()(sd, g2, z8)
    g3 = _tc_mid(acc2, g2, dinv, b2.reshape(1, -1), W3)
    acc3 = _sc_agg()(sd, g3, z8)
    out, h = _tc_final(acc3, g3, dinv, b3.reshape(1, -1),
                       Wc, bc.reshape(1, -1))
    return out[:N], h[:N]


# speedup vs baseline: 51.1108x; 1.0587x over previous
import functools

import jax
import jax.numpy as jnp
from jax import lax
from jax.experimental import pallas as pl
from jax.experimental.pallas import tpu as pltpu
from jax.experimental.pallas import tpu_sc as plsc

N = 10000
D = 128
E = 320000
NC, NS = 2, 16
NW = NC * NS
NP = 10240
RPW = NP // NS
H = 8
CHUNK = 128
CPW = 80
M = NW * CPW
EP = M * CHUNK
SETS = 8

BLK = 1024
GRID = NP // BLK
BLKF = 1000
GRIDF = N // BLKF


def _sc_mesh():
    return plsc.VectorSubcoreMesh(
        core_axis_name="c", subcore_axis_name="s",
        num_cores=NC, num_subcores=NS)


_SC_PARAMS = pltpu.CompilerParams(use_tc_tiling_on_sc=False)


@functools.lru_cache(maxsize=None)
def _sc_agg():
    @functools.partial(
        pl.kernel,
        out_type=jax.ShapeDtypeStruct((NC, NP, H), jnp.float32),
        mesh=_sc_mesh(),
        scratch_types=(
            [pltpu.VMEM((2, CHUNK), jnp.int32) for _ in range(SETS)]
            + [pltpu.VMEM((CHUNK, H), jnp.float32) for _ in range(SETS)]
            + [pltpu.SemaphoreType.DMA] * (2 * SETS)
            + [pltpu.VMEM_SHARED((NP, H), jnp.float32)]
        ),
        compiler_params=_SC_PARAMS,
    )
    def agg(sd_hbm, g_hbm, z_hbm, out_hbm, *scr):
        sdb = scr[:SETS]
        rows = scr[SETS:2 * SETS]
        gsem = scr[2 * SETS:3 * SETS]
        ssem = scr[3 * SETS:4 * SETS]
        acc = scr[4 * SETS]
        c = lax.axis_index("c")
        s = lax.axis_index("s")
        w = c * NS + s
        pltpu.sync_copy(z_hbm.at[pl.ds(s * RPW, RPW)],
                        acc.at[pl.ds(s * RPW, RPW)])
        plsc.subcore_barrier()

        @pl.loop(0, CPW, step=SETS)
        def body(j):
            descs = []
            for k in range(SETS):
                @pl.when(j > 0)
                def _(k=k):
                    pltpu.make_async_copy(
                        rows[k], acc.at[sdb[k].at[1]], ssem[k]).wait()
                descs.append(pltpu.async_copy(
                    sd_hbm.at[:, w * CPW + j + k], sdb[k], gsem[k]))
            for k in range(SETS):
                descs[k].wait()
                pltpu.async_copy(g_hbm.at[sdb[k].at[0]], rows[k], gsem[k])
            for k in range(SETS):
                pltpu.make_async_copy(
                    g_hbm.at[sdb[k].at[0]], rows[k], gsem[k]).wait()
                pltpu.async_copy(
                    rows[k], acc.at[sdb[k].at[1]], ssem[k], add=True)

        for k in range(SETS):
            pltpu.make_async_copy(
                rows[k], acc.at[sdb[k].at[1]], ssem[k]).wait()
        plsc.subcore_barrier()
        pltpu.sync_copy(acc.at[pl.ds(s * RPW, RPW)],
                        out_hbm.at[c, pl.ds(s * RPW, RPW)])

    return agg


@functools.lru_cache(maxsize=None)
def _sc_deg():
    @functools.partial(
        pl.kernel,
        out_type=jax.ShapeDtypeStruct((NC, NP, H), jnp.float32),
        mesh=_sc_mesh(),
        scratch_types=(
            [pltpu.VMEM((CHUNK,), jnp.int32) for _ in range(SETS)]
            + [pltpu.SemaphoreType.DMA] * SETS
            + [pltpu.VMEM((CHUNK, H), jnp.float32),
               pltpu.VMEM_SHARED((NP, H), jnp.float32)]
        ),
        compiler_params=_SC_PARAMS,
    )
    def deg(sd_hbm, ones_hbm, z_hbm, out_hbm, *scr):
        sdb = scr[:SETS]
        ssem = scr[SETS:2 * SETS]
        ones_v = scr[2 * SETS]
        acc = scr[2 * SETS + 1]
        c = lax.axis_index("c")
        s = lax.axis_index("s")
        w = c * NS + s
        pltpu.sync_copy(ones_hbm, ones_v)
        pltpu.sync_copy(z_hbm.at[pl.ds(s * RPW, RPW)],
                        acc.at[pl.ds(s * RPW, RPW)])
        plsc.subcore_barrier()

        @pl.loop(0, CPW, step=SETS)
        def body(j):
            descs = []
            for k in range(SETS):
                @pl.when(j > 0)
                def _(k=k):
                    pltpu.make_async_copy(
                        ones_v, acc.at[sdb[k]], ssem[k]).wait()
                descs.append(pltpu.async_copy(
                    sd_hbm.at[1, w * CPW + j + k], sdb[k], ssem[k]))
            for k in range(SETS):
                descs[k].wait()
                pltpu.async_copy(
                    ones_v, acc.at[sdb[k]], ssem[k], add=True)

        for k in range(SETS):
            pltpu.make_async_copy(
                ones_v, acc.at[sdb[k]], ssem[k]).wait()
        plsc.subcore_barrier()
        pltpu.sync_copy(acc.at[pl.ds(s * RPW, RPW)],
                        out_hbm.at[c, pl.ds(s * RPW, RPW)])

    return deg


def _tc_matmul1(xp, W1):
    def body(x_ref, w_ref, h_ref):
        h_ref[...] = jnp.dot(x_ref[...], w_ref[...],
                             preferred_element_type=jnp.float32)

    return pl.pallas_call(
        body,
        grid=(GRID,),
        in_specs=[
            pl.BlockSpec((BLK, D), lambda i: (i, 0)),
            pl.BlockSpec((D, 4), lambda i: (0, 0)),
        ],
        out_specs=pl.BlockSpec((BLK, 4), lambda i: (i, 0)),
        out_shape=jax.ShapeDtypeStruct((NP, 4), jnp.float32),
    )(xp, W1)


def _tc_scale1(h1, degs):
    def body(h_ref, deg_ref, g_ref, dinv_ref):
        deg = deg_ref[0, :, 0:1] + deg_ref[1, :, 0:1] + 1.0
        dinv = lax.rsqrt(deg)
        g_ref[...] = jnp.concatenate(
            [h_ref[...] * dinv, jnp.zeros((NP, H - 4), jnp.float32)], axis=1)
        dinv_ref[...] = dinv

    return pl.pallas_call(
        body,
        in_specs=[
            pl.BlockSpec((NP, 4), lambda: (0, 0)),
            pl.BlockSpec((NC, NP, H), lambda: (0, 0, 0)),
        ],
        out_specs=[
            pl.BlockSpec((NP, H), lambda: (0, 0)),
            pl.BlockSpec((NP, 1), lambda: (0, 0)),
        ],
        out_shape=[
            jax.ShapeDtypeStruct((NP, H), jnp.float32),
            jax.ShapeDtypeStruct((NP, 1), jnp.float32),
        ],
    )(h1, degs)


def _tc_mid(acc, g, dinv, b, W):
    hin = W.shape[0]
    hout = W.shape[1]

    def body(acc_ref, g_ref, dinv_ref, b_ref, w_ref, out_ref):
        dv = dinv_ref[...]
        t = ((acc_ref[0, :, :hin] + acc_ref[1, :, :hin] + g_ref[:, :hin]) * dv
             + b_ref[...])
        xx = jnp.tanh(t)
        h = jnp.dot(xx, w_ref[...], preferred_element_type=jnp.float32)
        out_ref[...] = jnp.concatenate(
            [h * dv, jnp.zeros((NP, H - hout), jnp.float32)], axis=1)

    return pl.pallas_call(
        body,
        in_specs=[
            pl.BlockSpec((NC, NP, H), lambda: (0, 0, 0)),
            pl.BlockSpec((NP, H), lambda: (0, 0)),
            pl.BlockSpec((NP, 1), lambda: (0, 0)),
            pl.BlockSpec((1, hin), lambda: (0, 0)),
            pl.BlockSpec((hin, hout), lambda: (0, 0)),
        ],
        out_specs=pl.BlockSpec((NP, H), lambda: (0, 0)),
        out_shape=jax.ShapeDtypeStruct((NP, H), jnp.float32),
    )(acc, g, dinv, b, W)


def _tc_final(acc, g, dinv, b, Wc, bc):
    hin = Wc.shape[0]
    c = Wc.shape[1]

    def body(acc_ref, g_ref, dinv_ref, b_ref, wc_ref, bc_ref,
             out_ref, h_ref):
        dv = dinv_ref[...]
        t = ((acc_ref[0, :, :hin] + acc_ref[1, :, :hin] + g_ref[:, :hin]) * dv
             + b_ref[...])
        x3 = jnp.tanh(t)
        out_ref[...] = jnp.dot(x3, wc_ref[...],
                               preferred_element_type=jnp.float32) + bc_ref[...]
        h_ref[...] = x3

    return pl.pallas_call(
        body,
        grid=(GRIDF,),
        in_specs=[
            pl.BlockSpec((NC, BLKF, H), lambda i: (0, i, 0)),
            pl.BlockSpec((BLKF, H), lambda i: (i, 0)),
            pl.BlockSpec((BLKF, 1), lambda i: (i, 0)),
            pl.BlockSpec((1, hin), lambda i: (0, 0)),
            pl.BlockSpec((hin, c), lambda i: (0, 0)),
            pl.BlockSpec((1, c), lambda i: (0, 0)),
        ],
        out_specs=[
            pl.BlockSpec((BLKF, c), lambda i: (i, 0)),
            pl.BlockSpec((BLKF, hin), lambda i: (i, 0)),
        ],
        out_shape=[
            jax.ShapeDtypeStruct((N, c), jnp.float32),
            jax.ShapeDtypeStruct((N, hin), jnp.float32),
        ],
    )(acc, g, dinv, b, Wc, bc)


def kernel(x, edge_index, W1, b1, W2, b2, W3, b3, Wc, bc):
    epw = E // NW
    ppw = CPW * CHUNK - epw
    pad_src = jnp.broadcast_to(
        N + 112 + (jnp.arange(ppw, dtype=jnp.int32) % 128), (NW, ppw))
    pad_dst = jnp.broadcast_to(
        N + 112 + ((jnp.arange(ppw, dtype=jnp.int32) + 64) % 128), (NW, ppw))
    srcp = jnp.concatenate(
        [edge_index[0].reshape(NW, epw), pad_src], axis=1).reshape(M, CHUNK)
    dstp = jnp.concatenate(
        [edge_index[1].reshape(NW, epw), pad_dst], axis=1).reshape(M, CHUNK)
    sd = jnp.stack([srcp, dstp], axis=0)
    xp = jnp.pad(x, ((0, NP - N), (0, 0)))
    z8 = jnp.zeros((NP, H), jnp.float32)
    ones = jnp.ones((CHUNK, H), jnp.float32)

    h1 = _tc_matmul1(xp, W1)
    degs = _sc_deg()(sd, ones, z8)
    g1, dinv = _tc_scale1(h1, degs)
    acc1 = _sc_agg()(sd, g1, z8)
    g2 = _tc_mid(acc1, g1, dinv, b1.reshape(1, -1), W2)
    acc2 = _sc_agg()(sd, g2, z8)
    g3 = _tc_mid(acc2, g2, dinv, b2.reshape(1, -1), W3)
    acc3 = _sc_agg()(sd, g3, z8)
    out, h = _tc_final(acc3, g3, dinv, b3.reshape(1, -1),
                       Wc, bc.reshape(1, -1))
    return out, h


# speedup vs baseline: 54.2100x; 1.0606x over previous
import functools

import jax
import jax.numpy as jnp
from jax import lax
from jax.experimental import pallas as pl
from jax.experimental.pallas import tpu as pltpu
from jax.experimental.pallas import tpu_sc as plsc

N = 10000
D = 128
E = 320000
NC, NS = 2, 16
NW = NC * NS
NP = 10240
RPW = NP // NS
H = 8
CHUNK = 128
CPW = 80
M = NW * CPW
EP = M * CHUNK
SETS = 8

BLK = 1024
GRID = NP // BLK
BLKF = 1000
GRIDF = N // BLKF


def _sc_mesh():
    return plsc.VectorSubcoreMesh(
        core_axis_name="c", subcore_axis_name="s",
        num_cores=NC, num_subcores=NS)


_SC_PARAMS = pltpu.CompilerParams(use_tc_tiling_on_sc=False)


@functools.lru_cache(maxsize=None)
def _sc_agg():
    @functools.partial(
        pl.kernel,
        out_type=jax.ShapeDtypeStruct((NC, NP, H), jnp.float32),
        mesh=_sc_mesh(),
        scratch_types=(
            [pltpu.VMEM((2, CHUNK), jnp.int32) for _ in range(SETS)]
            + [pltpu.VMEM((CHUNK, H), jnp.float32) for _ in range(SETS)]
            + [pltpu.SemaphoreType.DMA] * (2 * SETS)
            + [pltpu.VMEM_SHARED((NP, H), jnp.float32)]
        ),
        compiler_params=_SC_PARAMS,
    )
    def agg(sd_hbm, g_hbm, z_hbm, out_hbm, *scr):
        sdb = scr[:SETS]
        rows = scr[SETS:2 * SETS]
        gsem = scr[2 * SETS:3 * SETS]
        ssem = scr[3 * SETS:4 * SETS]
        acc = scr[4 * SETS]
        c = lax.axis_index("c")
        s = lax.axis_index("s")
        w = c * NS + s
        pltpu.sync_copy(z_hbm.at[pl.ds(s * RPW, RPW)],
                        acc.at[pl.ds(s * RPW, RPW)])
        plsc.subcore_barrier()

        @pl.loop(0, CPW, step=SETS)
        def body(j):
            descs = []
            for k in range(SETS):
                @pl.when(j > 0)
                def _(k=k):
                    pltpu.make_async_copy(
                        rows[k], acc.at[sdb[k].at[1]], ssem[k]).wait()
                descs.append(pltpu.async_copy(
                    sd_hbm.at[:, w * CPW + j + k], sdb[k], gsem[k]))
            for k in range(SETS):
                descs[k].wait()
                pltpu.async_copy(g_hbm.at[sdb[k].at[0]], rows[k], gsem[k])
            for k in range(SETS):
                pltpu.make_async_copy(
                    g_hbm.at[sdb[k].at[0]], rows[k], gsem[k]).wait()
                pltpu.async_copy(
                    rows[k], acc.at[sdb[k].at[1]], ssem[k], add=True)

        for k in range(SETS):
            pltpu.make_async_copy(
                rows[k], acc.at[sdb[k].at[1]], ssem[k]).wait()
        plsc.subcore_barrier()
        pltpu.sync_copy(acc.at[pl.ds(s * RPW, RPW)],
                        out_hbm.at[c, pl.ds(s * RPW, RPW)])

    return agg


@functools.lru_cache(maxsize=None)
def _sc_deg():
    @functools.partial(
        pl.kernel,
        out_type=jax.ShapeDtypeStruct((NC, NP, H), jnp.float32),
        mesh=_sc_mesh(),
        scratch_types=(
            [pltpu.VMEM((CHUNK,), jnp.int32) for _ in range(SETS)]
            + [pltpu.SemaphoreType.DMA] * SETS
            + [pltpu.VMEM((CHUNK, H), jnp.float32),
               pltpu.VMEM_SHARED((NP, H), jnp.float32)]
        ),
        compiler_params=_SC_PARAMS,
    )
    def deg(sd_hbm, ones_hbm, z_hbm, out_hbm, *scr):
        sdb = scr[:SETS]
        ssem = scr[SETS:2 * SETS]
        ones_v = scr[2 * SETS]
        acc = scr[2 * SETS + 1]
        c = lax.axis_index("c")
        s = lax.axis_index("s")
        w = c * NS + s
        pltpu.sync_copy(ones_hbm, ones_v)
        pltpu.sync_copy(z_hbm.at[pl.ds(s * RPW, RPW)],
                        acc.at[pl.ds(s * RPW, RPW)])
        plsc.subcore_barrier()

        @pl.loop(0, CPW, step=SETS)
        def body(j):
            descs = []
            for k in range(SETS):
                @pl.when(j > 0)
                def _(k=k):
                    pltpu.make_async_copy(
                        ones_v, acc.at[sdb[k]], ssem[k]).wait()
                descs.append(pltpu.async_copy(
                    sd_hbm.at[1, w * CPW + j + k], sdb[k], ssem[k]))
            for k in range(SETS):
                descs[k].wait()
                pltpu.async_copy(
                    ones_v, acc.at[sdb[k]], ssem[k], add=True)

        for k in range(SETS):
            pltpu.make_async_copy(
                ones_v, acc.at[sdb[k]], ssem[k]).wait()
        plsc.subcore_barrier()
        pltpu.sync_copy(acc.at[pl.ds(s * RPW, RPW)],
                        out_hbm.at[c, pl.ds(s * RPW, RPW)])

    return deg


def _tc_matmul1(x, W1):
    def body(x_ref, w_ref, h_ref):
        h_ref[...] = jnp.dot(x_ref[...], w_ref[...],
                             preferred_element_type=jnp.float32)

    return pl.pallas_call(
        body,
        grid=(GRIDF,),
        in_specs=[
            pl.BlockSpec((BLKF, D), lambda i: (i, 0)),
            pl.BlockSpec((D, 4), lambda i: (0, 0)),
        ],
        out_specs=pl.BlockSpec((BLKF, 4), lambda i: (i, 0)),
        out_shape=jax.ShapeDtypeStruct((N, 4), jnp.float32),
    )(x, W1)


def _tc_scale1(h1, degs):
    def body(h_ref, deg_ref, g_ref, dinv_ref):
        deg = deg_ref[0, :, 0:1] + deg_ref[1, :, 0:1] + 1.0
        dinv = lax.rsqrt(deg)
        g = h_ref[...] * dinv[:N]
        g = jnp.concatenate([g, jnp.zeros((NP - N, 4), jnp.float32)], axis=0)
        g_ref[...] = jnp.concatenate(
            [g, jnp.zeros((NP, H - 4), jnp.float32)], axis=1)
        dinv_ref[...] = dinv

    return pl.pallas_call(
        body,
        in_specs=[
            pl.BlockSpec((N, 4), lambda: (0, 0)),
            pl.BlockSpec((NC, NP, H), lambda: (0, 0, 0)),
        ],
        out_specs=[
            pl.BlockSpec((NP, H), lambda: (0, 0)),
            pl.BlockSpec((NP, 1), lambda: (0, 0)),
        ],
        out_shape=[
            jax.ShapeDtypeStruct((NP, H), jnp.float32),
            jax.ShapeDtypeStruct((NP, 1), jnp.float32),
        ],
    )(h1, degs)


def _tc_mid(acc, g, dinv, b, W):
    hin = W.shape[0]
    hout = W.shape[1]

    def body(acc_ref, g_ref, dinv_ref, b_ref, w_ref, out_ref):
        dv = dinv_ref[...]
        t = ((acc_ref[0, :, :hin] + acc_ref[1, :, :hin] + g_ref[:, :hin]) * dv
             + b_ref[...])
        xx = jnp.tanh(t)
        h = jnp.dot(xx, w_ref[...], preferred_element_type=jnp.float32)
        out_ref[...] = jnp.concatenate(
            [h * dv, jnp.zeros((NP, H - hout), jnp.float32)], axis=1)

    return pl.pallas_call(
        body,
        in_specs=[
            pl.BlockSpec((NC, NP, H), lambda: (0, 0, 0)),
            pl.BlockSpec((NP, H), lambda: (0, 0)),
            pl.BlockSpec((NP, 1), lambda: (0, 0)),
            pl.BlockSpec((1, hin), lambda: (0, 0)),
            pl.BlockSpec((hin, hout), lambda: (0, 0)),
        ],
        out_specs=pl.BlockSpec((NP, H), lambda: (0, 0)),
        out_shape=jax.ShapeDtypeStruct((NP, H), jnp.float32),
    )(acc, g, dinv, b, W)


def _tc_final(acc, g, dinv, b, Wc, bc):
    hin = Wc.shape[0]
    c = Wc.shape[1]

    def body(acc_ref, g_ref, dinv_ref, b_ref, wc_ref, bc_ref,
             out_ref, h_ref):
        dv = dinv_ref[...]
        t = ((acc_ref[0, :, :hin] + acc_ref[1, :, :hin] + g_ref[:, :hin]) * dv
             + b_ref[...])
        x3 = jnp.tanh(t)
        out_ref[...] = jnp.dot(x3, wc_ref[...],
                               preferred_element_type=jnp.float32) + bc_ref[...]
        h_ref[...] = x3

    return pl.pallas_call(
        body,
        grid=(GRIDF,),
        in_specs=[
            pl.BlockSpec((NC, BLKF, H), lambda i: (0, i, 0)),
            pl.BlockSpec((BLKF, H), lambda i: (i, 0)),
            pl.BlockSpec((BLKF, 1), lambda i: (i, 0)),
            pl.BlockSpec((1, hin), lambda i: (0, 0)),
            pl.BlockSpec((hin, c), lambda i: (0, 0)),
            pl.BlockSpec((1, c), lambda i: (0, 0)),
        ],
        out_specs=[
            pl.BlockSpec((BLKF, c), lambda i: (i, 0)),
            pl.BlockSpec((BLKF, hin), lambda i: (i, 0)),
        ],
        out_shape=[
            jax.ShapeDtypeStruct((N, c), jnp.float32),
            jax.ShapeDtypeStruct((N, hin), jnp.float32),
        ],
    )(acc, g, dinv, b, Wc, bc)


def kernel(x, edge_index, W1, b1, W2, b2, W3, b3, Wc, bc):
    epw = E // NW
    ppw = CPW * CHUNK - epw
    pad_src = N + 112 + (jnp.arange(ppw, dtype=jnp.int32) % 128)
    pad_dst = N + 112 + ((jnp.arange(ppw, dtype=jnp.int32) + 64) % 128)
    padblock = jnp.broadcast_to(
        jnp.stack([pad_src, pad_dst])[:, None, :], (2, NW, ppw))
    sd = jnp.concatenate(
        [edge_index.reshape(2, NW, epw), padblock],
        axis=2).reshape(2, M, CHUNK)
    z8 = jnp.zeros((NP, H), jnp.float32)
    ones = jnp.ones((CHUNK, H), jnp.float32)

    h1 = _tc_matmul1(x, W1)
    degs = _sc_deg()(sd, ones, z8)
    g1, dinv = _tc_scale1(h1, degs)
    acc1 = _sc_agg()(sd, g1, z8)
    g2 = _tc_mid(acc1, g1, dinv, b1.reshape(1, -1), W2)
    acc2 = _sc_agg()(sd, g2, z8)
    g3 = _tc_mid(acc2, g2, dinv, b2.reshape(1, -1), W3)
    acc3 = _sc_agg()(sd, g3, z8)
    out, h = _tc_final(acc3, g3, dinv, b3.reshape(1, -1),
                       Wc, bc.reshape(1, -1))
    return out, h
